# Initial kernel scaffold; baseline (speedup 1.0000x reference)
import jax
import jax.numpy as jnp
from jax.experimental import pallas as pl


def kernel(hidden_states, ln1_weight, ln1_bias, ln2_weight, ln2_bias, qkv_weight, proj_weight, router_weight, moe_w1, moe_w2):
    raise NotImplementedError("write your pallas kernel here")


# speedup vs baseline: 1.6174x; 1.6174x over previous
import functools

import jax
import jax.numpy as jnp
from jax import lax
from jax.experimental import pallas as pl
from jax.experimental.pallas import tpu as pltpu
from jax.experimental.pallas import tpu_sc as plsc

S, H, NH, HD, E, FF = 2048, 2048, 16, 128, 8, 2048
T = 512
NT = 15
P = NT * T
NFB = 4
BF = FF // NFB
NEG = -1e30
SCALE = 1.0 / (HD ** 0.5)


def _a1_body(x_ref, lnw_ref, lnb_ref, w_ref, out_ref, ln_scr):
    @pl.when(pl.program_id(0) == 0)
    def _():
        x = x_ref[...]
        m = jnp.mean(x, axis=1, keepdims=True)
        v = jnp.mean((x - m) ** 2, axis=1, keepdims=True)
        ln_scr[...] = (x - m) / jnp.sqrt(v + 1e-5) * lnw_ref[...] + lnb_ref[...]

    out_ref[...] = lax.dot_general(
        ln_scr[...], w_ref[...], (((1,), (1,)), ((), ())),
        preferred_element_type=jnp.float32)


def _a1(x2d, lnw, lnb, qkv_w):
    return pl.pallas_call(
        _a1_body,
        grid=(12,),
        in_specs=[
            pl.BlockSpec((S, H), lambda j: (0, 0)),
            pl.BlockSpec((1, H), lambda j: (0, 0)),
            pl.BlockSpec((1, H), lambda j: (0, 0)),
            pl.BlockSpec((512, H), lambda j: (j, 0)),
        ],
        out_specs=pl.BlockSpec((S, 512), lambda j: (0, j)),
        out_shape=jax.ShapeDtypeStruct((S, 3 * H), jnp.float32),
        scratch_shapes=[pltpu.VMEM((S, H), jnp.float32)],
    )(x2d, lnw.reshape(1, H), lnb.reshape(1, H), qkv_w)


def _a2_body(q_ref, k_ref, v_ref, o_ref):
    iq = pl.program_id(1)
    BQ = 512
    s = lax.dot_general(q_ref[...], k_ref[...], (((1,), (1,)), ((), ())),
                        preferred_element_type=jnp.float32) * SCALE
    rows = iq * BQ + lax.broadcasted_iota(jnp.int32, (BQ, S), 0)
    cols = lax.broadcasted_iota(jnp.int32, (BQ, S), 1)
    s = jnp.where(cols <= rows, s, NEG)
    m = jnp.max(s, axis=1, keepdims=True)
    p = jnp.exp(s - m)
    l = jnp.sum(p, axis=1, keepdims=True)
    o = lax.dot_general(p, v_ref[...], (((1,), (0,)), ((), ())),
                        preferred_element_type=jnp.float32)
    o_ref[...] = o / l


def _a2(qkv):
    return pl.pallas_call(
        _a2_body,
        grid=(NH, 4),
        in_specs=[
            pl.BlockSpec((512, HD), lambda h, iq: (iq, 3 * h)),
            pl.BlockSpec((S, HD), lambda h, iq: (0, 3 * h + 1)),
            pl.BlockSpec((S, HD), lambda h, iq: (0, 3 * h + 2)),
        ],
        out_specs=pl.BlockSpec((512, HD), lambda h, iq: (iq, h)),
        out_shape=jax.ShapeDtypeStruct((S, H), jnp.float32),
    )(qkv, qkv, qkv)


def _a3_body(attn_ref, x_ref, proj_ref, rw_ref, lnw_ref, lnb_ref,
             h2_ref, flat_ref, lt_ref):
    po = lax.dot_general(attn_ref[...], proj_ref[...], (((1,), (1,)), ((), ())),
                         preferred_element_type=jnp.float32)
    h2 = x_ref[...] + po
    h2_ref[...] = h2
    m = jnp.mean(h2, axis=1, keepdims=True)
    v = jnp.mean((h2 - m) ** 2, axis=1, keepdims=True)
    flat = (h2 - m) / jnp.sqrt(v + 1e-5) * lnw_ref[...] + lnb_ref[...]
    flat_ref[...] = flat
    lt_ref[...] = lax.dot_general(rw_ref[...], flat, (((0,), (1,)), ((), ())),
                                  preferred_element_type=jnp.float32)


def _a3(attn2d, x2d, proj_w, router_w, lnw, lnb):
    return pl.pallas_call(
        _a3_body,
        grid=(4,),
        in_specs=[
            pl.BlockSpec((512, H), lambda i: (i, 0)),
            pl.BlockSpec((512, H), lambda i: (i, 0)),
            pl.BlockSpec((H, H), lambda i: (0, 0)),
            pl.BlockSpec((H, E), lambda i: (0, 0)),
            pl.BlockSpec((1, H), lambda i: (0, 0)),
            pl.BlockSpec((1, H), lambda i: (0, 0)),
        ],
        out_specs=[
            pl.BlockSpec((512, H), lambda i: (i, 0)),
            pl.BlockSpec((512, H), lambda i: (i, 0)),
            pl.BlockSpec((E, 512), lambda i: (0, i)),
        ],
        out_shape=[
            jax.ShapeDtypeStruct((S, H), jnp.float32),
            jax.ShapeDtypeStruct((S, H), jnp.float32),
            jax.ShapeDtypeStruct((E, S), jnp.float32),
        ],
    )(attn2d, x2d, proj_w, router_w, lnw.reshape(1, H), lnb.reshape(1, H))


_MESH = plsc.VectorSubcoreMesh(core_axis_name="c", subcore_axis_name="s")
_NGALL = S // 16
_TCH = S // 32
_SCH = P // 32

_SC_PARAMS = pltpu.CompilerParams(needs_layout_passes=False)


def _plan_body(lt_hbm, pos1_hbm, pos2_hbm, p1_hbm, p2_hbm, eids_hbm, src_hbm,
               lg, i1a, i2a, pv1, pv2, posa1, posa2, evr, myst):
    c = lax.axis_index("c")
    u = lax.axis_index("s") * 2 + c
    lanes = lax.iota(jnp.int32, 16)
    zeros = jnp.zeros((16,), jnp.int32)
    ones = jnp.full((16,), 1, jnp.int32)
    big = jnp.full((16,), 999, jnp.int32)
    negv = jnp.full((16,), NEG, jnp.float32)

    pltpu.sync_copy(lt_hbm, lg)

    def rbody(g, cnt):
        g16 = g * 16
        vs = [lg[pl.ds(e * S + g16, 16)] for e in range(E)]
        m1 = jnp.maximum(vs[0], vs[1])
        for e in range(2, E):
            m1 = jnp.maximum(m1, vs[e])
        i1 = big
        for e in range(E):
            ev = jnp.full((16,), e, jnp.int32)
            i1 = jnp.minimum(i1, jnp.where(vs[e] == m1, ev, big))
        m2 = negv
        for e in range(E):
            ev = jnp.full((16,), e, jnp.int32)
            m2 = jnp.maximum(m2, jnp.where(i1 == ev, negv, vs[e]))
        i2 = big
        for e in range(E):
            ev = jnp.full((16,), e, jnp.int32)
            cand = jnp.where(vs[e] == m2, ev, big)
            i2 = jnp.minimum(i2, jnp.where(i1 == ev, big, cand))
        pp1 = 1.0 / (1.0 + jnp.exp(m2 - m1))
        i1a[pl.ds(g16, 16)] = i1
        i2a[pl.ds(g16, 16)] = i2
        pv1[pl.ds(g16, 16)] = pp1
        pv2[pl.ds(g16, 16)] = 1.0 - pp1
        for e in range(E):
            ev = jnp.full((16,), e, jnp.int32)
            c1 = jnp.sum(jnp.where(i1 == ev, ones, zeros))
            ev2 = jnp.full((16,), e, jnp.int32)
            c2 = jnp.sum(jnp.where(i2 == ev2, ones, zeros))
            cs = jnp.full((16,), c1 + c2, jnp.int32)
            ev3 = jnp.full((16,), e, jnp.int32)
            cnt = cnt + jnp.where(lanes == ev3, cs, zeros)
        return cnt

    cnt = lax.fori_loop(0, _NGALL, rbody, zeros)

    start = jnp.int32(0)
    starts = [None] * E
    end_tile = [None] * E
    for e in range(E):
        ev = jnp.full((16,), e, jnp.int32)
        tot = jnp.sum(jnp.where(lanes == ev, cnt, zeros))
        cap = ((tot + (T - 1)) >> 9) << 9
        starts[e] = start
        end_tile[e] = (start + cap) >> 9
        start = start + cap
    tv = zeros
    for e in range(E):
        et = jnp.full((16,), end_tile[e], jnp.int32)
        tv = tv + jnp.where(lanes >= et, ones, zeros)
    evr[...] = jnp.minimum(tv, jnp.full((16,), E - 1, jnp.int32))

    @pl.when(u == 0)
    def _():
        pltpu.sync_copy(evr, eids_hbm)

    for q in range(_SCH // 16):
        myst[pl.ds(q * 16, 16)] = jnp.full((16,), S - 1, jnp.int32)

    ptr0 = zeros
    for e in range(E):
        ev = jnp.full((16,), e, jnp.int32)
        sv = jnp.full((16,), starts[e], jnp.int32)
        ptr0 = jnp.where(lanes == ev, sv, ptr0)

    lov = jnp.full((16,), u * _SCH, jnp.int32)
    hiv = jnp.full((16,), u * _SCH + _SCH, jnp.int32)

    def pbody(g, ptrv):
        g16 = g * 16
        tokseg = g16 + lanes
        for iv_ref, posv in ((i1a, posa1), (i2a, posa2)):
            iv = iv_ref[pl.ds(g16, 16)]
            pos = zeros
            for e in range(E):
                ev = jnp.full((16,), e, jnp.int32)
                mki = jnp.where(iv == ev, ones, zeros)
                rank = jnp.cumsum(mki) - ones
                pbase = jnp.sum(jnp.where(lanes == ev, ptrv, zeros))
                pbv = jnp.full((16,), pbase, jnp.int32)
                ev2 = jnp.full((16,), e, jnp.int32)
                pos = jnp.where(iv == ev2, pbv + rank, pos)
                csv = jnp.full((16,), jnp.sum(mki), jnp.int32)
                ev3 = jnp.full((16,), e, jnp.int32)
                ptrv = ptrv + jnp.where(lanes == ev3, csv, zeros)
            posv[pl.ds(g16, 16)] = pos
            inb = jnp.logical_and(pos >= lov, pos < hiv)
            plsc.store_scatter(myst, [pos - lov], tokseg, mask=inb)
        return ptrv

    lax.fori_loop(0, _NGALL, pbody, ptr0)

    tb = u * _TCH
    pltpu.sync_copy(posa1.at[pl.ds(tb, _TCH)], pos1_hbm.at[pl.ds(tb, _TCH)])
    pltpu.sync_copy(posa2.at[pl.ds(tb, _TCH)], pos2_hbm.at[pl.ds(tb, _TCH)])
    pltpu.sync_copy(pv1.at[pl.ds(tb, _TCH)], p1_hbm.at[pl.ds(tb, _TCH)])
    pltpu.sync_copy(pv2.at[pl.ds(tb, _TCH)], p2_hbm.at[pl.ds(tb, _TCH)])
    pltpu.sync_copy(myst, src_hbm.at[pl.ds(u * _SCH, _SCH)])


_plan = functools.partial(
    pl.kernel,
    mesh=_MESH,
    compiler_params=_SC_PARAMS,
    out_type=[
        jax.ShapeDtypeStruct((S,), jnp.int32),
        jax.ShapeDtypeStruct((S,), jnp.int32),
        jax.ShapeDtypeStruct((S,), jnp.float32),
        jax.ShapeDtypeStruct((S,), jnp.float32),
        jax.ShapeDtypeStruct((16,), jnp.int32),
        jax.ShapeDtypeStruct((P,), jnp.int32),
    ],
    scratch_types=[
        pltpu.VMEM((E * S,), jnp.float32),
        pltpu.VMEM((S,), jnp.int32),
        pltpu.VMEM((S,), jnp.int32),
        pltpu.VMEM((S,), jnp.float32),
        pltpu.VMEM((S,), jnp.float32),
        pltpu.VMEM((S,), jnp.int32),
        pltpu.VMEM((S,), jnp.int32),
        pltpu.VMEM((16,), jnp.int32),
        pltpu.VMEM((_SCH,), jnp.int32),
    ],
)(_plan_body)


_GC = P // 32
_GB = 48


def _gather_body(src_hbm, flat_hbm, disp_hbm, idxv, rows):
    c = lax.axis_index("c")
    wid = lax.axis_index("s") * 2 + c
    sbase = wid * _GC
    pltpu.sync_copy(src_hbm.at[pl.ds(sbase, _GC)], idxv)
    for b in range(_GC // _GB):
        pltpu.sync_copy(flat_hbm.at[idxv.at[pl.ds(b * _GB, _GB)]], rows)
        pltpu.sync_copy(rows, disp_hbm.at[pl.ds(sbase + b * _GB, _GB)])


_gather = functools.partial(
    pl.kernel,
    mesh=_MESH,
    compiler_params=_SC_PARAMS,
    out_type=jax.ShapeDtypeStruct((P, H), jnp.float32),
    scratch_types=[
        pltpu.VMEM((_GC,), jnp.int32),
        pltpu.VMEM((_GB, H), jnp.float32),
    ],
)(_gather_body)


def _moe_body(eids_ref, disp_ref, w1_ref, w2_ref, out_ref):
    del eids_ref
    h = lax.dot_general(disp_ref[...], w1_ref[0], (((1,), (0,)), ((), ())),
                        preferred_element_type=jnp.float32)
    g = 0.5 * h * (1.0 + lax.erf(h / 1.4142135623730951))
    contrib = lax.dot_general(g, w2_ref[0], (((1,), (0,)), ((), ())),
                              preferred_element_type=jnp.float32)

    @pl.when(pl.program_id(1) == 0)
    def _():
        out_ref[...] = contrib

    @pl.when(pl.program_id(1) != 0)
    def _():
        out_ref[...] += contrib


def _moe(eids, disp, w1, w2):
    grid_spec = pltpu.PrefetchScalarGridSpec(
        num_scalar_prefetch=1,
        grid=(NT, NFB),
        in_specs=[
            pl.BlockSpec((T, H), lambda j, f, eids: (j, 0)),
            pl.BlockSpec((1, H, BF), lambda j, f, eids: (eids[j], 0, f)),
            pl.BlockSpec((1, BF, H), lambda j, f, eids: (eids[j], f, 0)),
        ],
        out_specs=pl.BlockSpec((T, H), lambda j, f, eids: (j, 0)),
    )
    return pl.pallas_call(
        _moe_body,
        grid_spec=grid_spec,
        out_shape=jax.ShapeDtypeStruct((P, H), jnp.float32),
    )(eids, disp, w1, w2)


_CT = S // 32


def _combine_body(dout_hbm, h2_hbm, pos1_hbm, pos2_hbm, p1_hbm, p2_hbm,
                  out_hbm, pos1v, pos2v, pa, pb, r1, r2, ro):
    c = lax.axis_index("c")
    wid = lax.axis_index("s") * 2 + c
    base = wid * _CT
    pltpu.sync_copy(pos1_hbm.at[pl.ds(base, _CT)], pos1v)
    pltpu.sync_copy(pos2_hbm.at[pl.ds(base, _CT)], pos2v)
    pltpu.sync_copy(p1_hbm.at[pl.ds(base, _CT)], pa)
    pltpu.sync_copy(p2_hbm.at[pl.ds(base, _CT)], pb)
    for ch in range(_CT // 16):
        pltpu.sync_copy(dout_hbm.at[pos1v.at[pl.ds(ch * 16, 16)]], r1)
        pltpu.sync_copy(dout_hbm.at[pos2v.at[pl.ds(ch * 16, 16)]], r2)
        pltpu.sync_copy(h2_hbm.at[pl.ds(base + ch * 16, 16)], ro)
        pav = pa[pl.ds(ch * 16, 16)]
        pbv = pb[pl.ds(ch * 16, 16)]
        aa = [pav[t] for t in range(16)]
        bb = [pbv[t] for t in range(16)]

        def body(si, carry):
            off = si * 16
            for t in range(16):
                ro[t, pl.ds(off, 16)] = (ro[t, pl.ds(off, 16)]
                                         + aa[t] * r1[t, pl.ds(off, 16)]
                                         + bb[t] * r2[t, pl.ds(off, 16)])
            return carry

        lax.fori_loop(0, H // 16, body, jnp.int32(0))
        pltpu.sync_copy(ro, out_hbm.at[pl.ds(base + ch * 16, 16)])


_combine = functools.partial(
    pl.kernel,
    mesh=_MESH,
    compiler_params=_SC_PARAMS,
    out_type=jax.ShapeDtypeStruct((S, H), jnp.float32),
    scratch_types=[
        pltpu.VMEM((_CT,), jnp.int32),
        pltpu.VMEM((_CT,), jnp.int32),
        pltpu.VMEM((_CT,), jnp.float32),
        pltpu.VMEM((_CT,), jnp.float32),
        pltpu.VMEM((16, H), jnp.float32),
        pltpu.VMEM((16, H), jnp.float32),
        pltpu.VMEM((16, H), jnp.float32),
    ],
)(_combine_body)


def kernel(hidden_states, ln1_weight, ln1_bias, ln2_weight, ln2_bias,
           qkv_weight, proj_weight, router_weight, moe_w1, moe_w2):
    x2d = hidden_states.reshape(S, H)
    qkv = _a1(x2d, ln1_weight, ln1_bias, qkv_weight)
    attn2d = _a2(qkv)
    h2, flat, logitsT = _a3(attn2d, x2d, proj_weight, router_weight,
                            ln2_weight, ln2_bias)
    pos1, pos2, p1, p2, eids, srctok = _plan(logitsT.reshape(-1))
    disp = _gather(srctok, flat)
    dout = _moe(eids, disp, moe_w1, moe_w2)
    out2d = _combine(dout, h2, pos1, pos2, p1, p2)
    return out2d.reshape(S, 1, H)
